# baseline (speedup 1.0000x reference)
import functools

import jax
import jax.numpy as jnp
from jax import lax
from jax.experimental import pallas as pl
from jax.experimental.pallas import tpu as pltpu
from jax.experimental.pallas import tpu_sc as plsc

_B, _N, _CIN, _COUT, _K = 4, 4096, 64, 128, 16
_HID = _COUT // 2
_KP1 = _K + 1
_EPS = 1e-5
_ROWS = 512
_EPTS = 256
_CB = 2048


def _lrelu(x):
    return jnp.where(x >= 0, x, 0.2 * x)


def _pre_kernel(f_ref, wpre_ref, wsc_ref, p_ref, scp_ref, ps_ref, ss_ref):
    b = pl.program_id(0)
    c = pl.program_id(1)

    @pl.when(jnp.logical_and(b == 0, c == 0))
    def _():
        ps_ref[...] = jnp.zeros_like(ps_ref)
        ss_ref[...] = jnp.zeros_like(ss_ref)

    fb = f_ref[0]
    p = jnp.dot(wpre_ref[...], fb, preferred_element_type=jnp.float32)
    sc = jnp.dot(wsc_ref[...], fb, preferred_element_type=jnp.float32)
    p_ref[0] = p
    scp_ref[0] = sc
    ps_ref[...] += jnp.concatenate(
        [jnp.sum(p, axis=1, keepdims=True), jnp.sum(p * p, axis=1, keepdims=True)], axis=1)
    ss_ref[...] += jnp.concatenate(
        [jnp.sum(sc, axis=1, keepdims=True), jnp.sum(sc * sc, axis=1, keepdims=True)], axis=1)


def _pre_pass(features, W_pre, W_sc):
    nc = _N // _CB
    return pl.pallas_call(
        _pre_kernel,
        grid=(_B, nc),
        in_specs=[
            pl.BlockSpec((1, _CIN, _CB), lambda b, c: (b, 0, c)),
            pl.BlockSpec((_HID, _CIN), lambda b, c: (0, 0)),
            pl.BlockSpec((_COUT, _CIN), lambda b, c: (0, 0)),
        ],
        out_specs=[
            pl.BlockSpec((1, _HID, _CB), lambda b, c: (b, 0, c)),
            pl.BlockSpec((1, _COUT, _CB), lambda b, c: (b, 0, c)),
            pl.BlockSpec((_HID, 2), lambda b, c: (0, 0)),
            pl.BlockSpec((_COUT, 2), lambda b, c: (0, 0)),
        ],
        out_shape=[
            jax.ShapeDtypeStruct((_B, _HID, _N), jnp.float32),
            jax.ShapeDtypeStruct((_B, _COUT, _N), jnp.float32),
            jax.ShapeDtypeStruct((_HID, 2), jnp.float32),
            jax.ShapeDtypeStruct((_COUT, 2), jnp.float32),
        ],
    )(features, W_pre, W_sc)


def _knn_kernel(xq_ref, xt_ref, idx_ref, dist_ref):
    b = pl.program_id(0)
    xq = xq_ref[0]
    xt = xt_ref[0]
    sqa = jnp.sum(xt * xt, axis=0, keepdims=True)
    sqq = jnp.sum(xq * xq, axis=1, keepdims=True)
    mm = jnp.dot(xq.astype(jnp.bfloat16), xt.astype(jnp.bfloat16),
                 preferred_element_type=jnp.float32)
    d2 = jnp.maximum(sqq + sqa - 2.0 * mm, 0.0)

    iota = lax.broadcasted_iota(jnp.int32, (_ROWS, _N), 1)
    big = jnp.float32(jnp.inf)
    idxs, vals = [], []
    for _ in range(_KP1):
        m = jnp.min(d2, axis=1, keepdims=True)
        am = jnp.min(jnp.where(d2 == m, iota, _N), axis=1, keepdims=True)
        idxs.append(am)
        vals.append(m)
        d2 = jnp.where(iota == am, big, d2)
    idx_ref[0] = jnp.concatenate(idxs, axis=1) + b * _N
    dist_ref[0] = jnp.sqrt(jnp.concatenate(vals, axis=1))


def _knn_pass(xyz, xyzT):
    nr = _N // _ROWS
    return pl.pallas_call(
        _knn_kernel,
        grid=(_B, nr),
        in_specs=[
            pl.BlockSpec((1, _ROWS, 3), lambda b, r: (b, r, 0)),
            pl.BlockSpec((1, 3, _N), lambda b, r: (b, 0, 0)),
        ],
        out_specs=[
            pl.BlockSpec((1, _ROWS, _KP1), lambda b, r: (b, r, 0)),
            pl.BlockSpec((1, _ROWS, _KP1), lambda b, r: (b, r, 0)),
        ],
        out_shape=[
            jax.ShapeDtypeStruct((_B, _N, _KP1), jnp.int32),
            jax.ShapeDtypeStruct((_B, _N, _KP1), jnp.float32),
        ],
    )(xyz, xyzT)


def _fuv_kernel(p_ref, xt_ref, wn_ref, wcm_ref, wx_ref, s1_ref, t1_ref,
                u_ref, v_ref):
    p = p_ref[0]
    x = xt_ref[0]
    f = _lrelu(p * s1_ref[...] + t1_ref[...])
    wx = wx_ref[...]
    gx = (wx[:, 0:1] * x[0:1, :] + wx[:, 1:2] * x[1:2, :]
          + wx[:, 2:3] * x[2:3, :])
    u_ref[0] = jnp.dot(wn_ref[...], f, preferred_element_type=jnp.float32) + gx
    v_ref[0] = jnp.dot(wcm_ref[...], f, preferred_element_type=jnp.float32) - gx


def _fuv_pass(P, xyzT, Wn, Wcm, Wx, s1, t1):
    nc = _N // _CB
    return pl.pallas_call(
        _fuv_kernel,
        grid=(_B, nc),
        in_specs=[
            pl.BlockSpec((1, _HID, _CB), lambda b, c: (b, 0, c)),
            pl.BlockSpec((1, 3, _CB), lambda b, c: (b, 0, c)),
            pl.BlockSpec((_COUT, _HID), lambda b, c: (0, 0)),
            pl.BlockSpec((_COUT, _HID), lambda b, c: (0, 0)),
            pl.BlockSpec((_COUT, 3), lambda b, c: (0, 0)),
            pl.BlockSpec((_HID, 1), lambda b, c: (0, 0)),
            pl.BlockSpec((_HID, 1), lambda b, c: (0, 0)),
        ],
        out_specs=[
            pl.BlockSpec((1, _COUT, _CB), lambda b, c: (b, 0, c)),
            pl.BlockSpec((1, _COUT, _CB), lambda b, c: (b, 0, c)),
        ],
        out_shape=[
            jax.ShapeDtypeStruct((_B, _COUT, _N), jnp.float32),
            jax.ShapeDtypeStruct((_B, _COUT, _N), jnp.float32),
        ],
    )(P, xyzT, Wn, Wcm, Wx, s1, t1)


_SC_CHUNK = 512


def _gather_rows(table, idx):
    E = idx.shape[0]
    info = plsc.get_sparse_core_info()
    nw = info.num_cores * info.num_subcores
    per_w = E // nw
    nchunk = per_w // _SC_CHUNK
    mesh = plsc.VectorSubcoreMesh(core_axis_name="c", subcore_axis_name="s")

    @functools.partial(
        pl.kernel,
        out_type=jax.ShapeDtypeStruct((E, _COUT), jnp.float32),
        mesh=mesh,
        scratch_types=[
            pltpu.VMEM((_SC_CHUNK,), jnp.int32),
            pltpu.VMEM((_SC_CHUNK, _COUT), jnp.float32),
            pltpu.SemaphoreType.DMA,
        ],
    )
    def sc_gather(table_hbm, idx_hbm, out_hbm, idx_v, rows_v, sem):
        wid = lax.axis_index("s") * info.num_cores + lax.axis_index("c")
        base = wid * per_w
        for c in range(nchunk):
            off = base + c * _SC_CHUNK
            pltpu.sync_copy(idx_hbm.at[pl.ds(off, _SC_CHUNK)], idx_v)
            pltpu.async_copy(table_hbm.at[idx_v], rows_v, sem).wait()
            pltpu.sync_copy(rows_v, out_hbm.at[pl.ds(off, _SC_CHUNK)])

    return sc_gather(table, idx)


def _edge_e1(g_ref, vt_ref, d_ref, wd_ref):
    u = g_ref[...].reshape(_EPTS, _K, _COUT)
    d = d_ref[...]
    vt = vt_ref[...]
    wd = wd_ref[...]
    return u + vt[:, None, :] + d[:, :, None] * wd[None, :, :]


def _stats1_kernel(g_ref, vt_ref, d_ref, wd_ref, s_ref):
    @pl.when(pl.program_id(0) == 0)
    def _():
        s_ref[...] = jnp.zeros_like(s_ref)

    e1 = _edge_e1(g_ref, vt_ref, d_ref, wd_ref)
    s_ref[...] += jnp.stack(
        [jnp.sum(e1, axis=(0, 1)), jnp.sum(e1 * e1, axis=(0, 1))], axis=0)


def _stats2_kernel(g_ref, vt_ref, d_ref, wd_ref, s2_ref, t2_ref, we2_ref,
                   s_ref):
    @pl.when(pl.program_id(0) == 0)
    def _():
        s_ref[...] = jnp.zeros_like(s_ref)

    e1 = _edge_e1(g_ref, vt_ref, d_ref, wd_ref)
    h1 = _lrelu(e1 * s2_ref[...][None, :, :] + t2_ref[...][None, :, :])
    e2 = jnp.dot(h1.reshape(_EPTS * _K, _COUT).astype(jnp.bfloat16),
                 we2_ref[...].astype(jnp.bfloat16),
                 preferred_element_type=jnp.float32)
    s_ref[...] += jnp.stack(
        [jnp.sum(e2, axis=0), jnp.sum(e2 * e2, axis=0)], axis=0)


def _final_kernel(g_ref, vt_ref, d_ref, wd_ref, s2_ref, t2_ref, we2_ref,
                  s3_ref, t3_ref, scp_ref, s4_ref, t4_ref, o_ref):
    e1 = _edge_e1(g_ref, vt_ref, d_ref, wd_ref)
    h1 = _lrelu(e1 * s2_ref[...][None, :, :] + t2_ref[...][None, :, :])
    e2 = jnp.dot(h1.reshape(_EPTS * _K, _COUT).astype(jnp.bfloat16),
                 we2_ref[...].astype(jnp.bfloat16),
                 preferred_element_type=jnp.float32)
    h2 = _lrelu(e2 * s3_ref[...] + t3_ref[...]).reshape(_EPTS, _K, _COUT)
    mx = jnp.max(h2, axis=1)
    sc = scp_ref[...] * s4_ref[...] + t4_ref[...]
    o_ref[...] = _lrelu(mx + sc)


def _edge_specs(extra):
    ne = (_B * _N) // _EPTS
    specs = [
        pl.BlockSpec((_EPTS * _K, _COUT), lambda i: (i, 0)),
        pl.BlockSpec((_EPTS, _COUT), lambda i: (i, 0)),
        pl.BlockSpec((_EPTS, _K), lambda i: (i, 0)),
        pl.BlockSpec((1, _COUT), lambda i: (0, 0)),
    ]
    specs += [pl.BlockSpec(s, lambda i: (0, 0)) for s in extra]
    return ne, specs


def _stats1_pass(G, vT, dist, wd):
    ne, specs = _edge_specs([])
    return pl.pallas_call(
        _stats1_kernel, grid=(ne,), in_specs=specs,
        out_specs=pl.BlockSpec((2, _COUT), lambda i: (0, 0)),
        out_shape=jax.ShapeDtypeStruct((2, _COUT), jnp.float32),
    )(G, vT, dist, wd)


def _stats2_pass(G, vT, dist, wd, s2, t2, We2T):
    ne, specs = _edge_specs([(1, _COUT), (1, _COUT), (_COUT, _COUT)])
    return pl.pallas_call(
        _stats2_kernel, grid=(ne,), in_specs=specs,
        out_specs=pl.BlockSpec((2, _COUT), lambda i: (0, 0)),
        out_shape=jax.ShapeDtypeStruct((2, _COUT), jnp.float32),
    )(G, vT, dist, wd, s2, t2, We2T)


def _final_pass(G, vT, dist, wd, s2, t2, We2T, s3, t3, SCpT, s4, t4):
    ne, specs = _edge_specs([(1, _COUT), (1, _COUT), (_COUT, _COUT),
                             (1, _COUT), (1, _COUT)])
    specs.append(pl.BlockSpec((_EPTS, _COUT), lambda i: (i, 0)))
    specs += [pl.BlockSpec((1, _COUT), lambda i: (0, 0))] * 2
    return pl.pallas_call(
        _final_kernel, grid=(ne,), in_specs=specs,
        out_specs=pl.BlockSpec((_EPTS, _COUT), lambda i: (i, 0)),
        out_shape=jax.ShapeDtypeStruct((_B * _N, _COUT), jnp.float32),
    )(G, vT, dist, wd, s2, t2, We2T, s3, t3, SCpT, s4, t4)


def _bn_coef(sums, m, g, b):
    mean = sums[:, 0] / m
    var = sums[:, 1] / m - mean * mean
    s = g / jnp.sqrt(var + _EPS)
    return s, b - mean * s


def _bn_coef_row(sums, m, g, b):
    mean = sums[0] / m
    var = sums[1] / m - mean * mean
    s = g / jnp.sqrt(var + _EPS)
    return s.reshape(1, _COUT), (b - mean * s).reshape(1, _COUT)


def kernel(xyz, features, W_pre, g1, b1, W_e1, g2, b2, W_e2, g3, b3,
           W_sc, g4, b4):
    xyzT = jnp.transpose(xyz, (0, 2, 1))

    P, SCp, psums, ssums = _pre_pass(features, W_pre, W_sc)
    s1, t1 = _bn_coef(psums, _B * _N, g1, b1)
    s4r = (g4 / jnp.sqrt(ssums[:, 1] / (_B * _N)
                         - (ssums[:, 0] / (_B * _N)) ** 2 + _EPS))
    t4r = b4 - (ssums[:, 0] / (_B * _N)) * s4r
    s4 = s4r.reshape(1, _COUT)
    t4 = t4r.reshape(1, _COUT)

    idxf, dist = _knn_pass(xyz, xyzT)
    idx_flat = idxf[:, :, 1:].reshape(-1)
    dist16 = dist[:, :, 1:].reshape(_B * _N, _K)

    Wc = W_e1[:, :_HID]
    Wn = W_e1[:, _HID:2 * _HID]
    Wx = W_e1[:, 2 * _HID:2 * _HID + 3]
    wd = W_e1[:, 2 * _HID + 3].reshape(1, _COUT)
    u, v = _fuv_pass(P, xyzT, Wn, Wc - Wn, Wx,
                     s1.reshape(_HID, 1), t1.reshape(_HID, 1))

    uT = u.transpose(0, 2, 1).reshape(_B * _N, _COUT)
    vT = v.transpose(0, 2, 1).reshape(_B * _N, _COUT)
    SCpT = SCp.transpose(0, 2, 1).reshape(_B * _N, _COUT)

    G = _gather_rows(uT, idx_flat)

    M = _B * _N * _K
    sums1 = _stats1_pass(G, vT, dist16, wd)
    s2, t2 = _bn_coef_row(sums1, M, g2, b2)
    sums2 = _stats2_pass(G, vT, dist16, wd, s2, t2, W_e2.T)
    s3, t3 = _bn_coef_row(sums2, M, g3, b3)

    out = _final_pass(G, vT, dist16, wd, s2, t2, W_e2.T, s3, t3, SCpT, s4, t4)
    return out.reshape(_B, _N, _COUT).transpose(0, 2, 1)


# speedup vs baseline: 1.0554x; 1.0554x over previous
import functools

import jax
import jax.numpy as jnp
from jax import lax
from jax.experimental import pallas as pl
from jax.experimental.pallas import tpu as pltpu
from jax.experimental.pallas import tpu_sc as plsc

_B, _N, _CIN, _COUT, _K = 4, 4096, 64, 128, 16
_HID = _COUT // 2
_KP1 = _K + 1
_EPS = 1e-5
_ROWS = 128
_EPTS = 256
_CB = 2048


def _lrelu(x):
    return jnp.where(x >= 0, x, 0.2 * x)


def _pre_kernel(f_ref, wpre_ref, wsc_ref, p_ref, scp_ref, ps_ref, ss_ref):
    b = pl.program_id(0)
    c = pl.program_id(1)

    @pl.when(jnp.logical_and(b == 0, c == 0))
    def _():
        ps_ref[...] = jnp.zeros_like(ps_ref)
        ss_ref[...] = jnp.zeros_like(ss_ref)

    fb = f_ref[0]
    p = jnp.dot(wpre_ref[...], fb, preferred_element_type=jnp.float32)
    sc = jnp.dot(wsc_ref[...], fb, preferred_element_type=jnp.float32)
    p_ref[0] = p
    scp_ref[0] = sc
    ps_ref[...] += jnp.concatenate(
        [jnp.sum(p, axis=1, keepdims=True), jnp.sum(p * p, axis=1, keepdims=True)], axis=1)
    ss_ref[...] += jnp.concatenate(
        [jnp.sum(sc, axis=1, keepdims=True), jnp.sum(sc * sc, axis=1, keepdims=True)], axis=1)


def _pre_pass(features, W_pre, W_sc):
    nc = _N // _CB
    return pl.pallas_call(
        _pre_kernel,
        grid=(_B, nc),
        in_specs=[
            pl.BlockSpec((1, _CIN, _CB), lambda b, c: (b, 0, c)),
            pl.BlockSpec((_HID, _CIN), lambda b, c: (0, 0)),
            pl.BlockSpec((_COUT, _CIN), lambda b, c: (0, 0)),
        ],
        out_specs=[
            pl.BlockSpec((1, _HID, _CB), lambda b, c: (b, 0, c)),
            pl.BlockSpec((1, _COUT, _CB), lambda b, c: (b, 0, c)),
            pl.BlockSpec((_HID, 2), lambda b, c: (0, 0)),
            pl.BlockSpec((_COUT, 2), lambda b, c: (0, 0)),
        ],
        out_shape=[
            jax.ShapeDtypeStruct((_B, _HID, _N), jnp.float32),
            jax.ShapeDtypeStruct((_B, _COUT, _N), jnp.float32),
            jax.ShapeDtypeStruct((_HID, 2), jnp.float32),
            jax.ShapeDtypeStruct((_COUT, 2), jnp.float32),
        ],
    )(features, W_pre, W_sc)


def _knn_kernel(xq_ref, xt_ref, idx_ref, dist_ref):
    b = pl.program_id(0)
    xq = xq_ref[0]
    xt = xt_ref[0]
    sqa = jnp.sum(xt * xt, axis=0, keepdims=True)
    sqq = jnp.sum(xq * xq, axis=1, keepdims=True)
    mm = jnp.dot(xq.astype(jnp.bfloat16), xt.astype(jnp.bfloat16),
                 preferred_element_type=jnp.float32)
    d2 = jnp.maximum(sqq + sqa - 2.0 * mm, 0.0)

    iota = lax.broadcasted_iota(jnp.int32, (_ROWS, _N), 1)
    big = jnp.float32(jnp.inf)
    idxs, vals = [], []
    for _ in range(_KP1):
        m = jnp.min(d2, axis=1, keepdims=True)
        am = jnp.min(jnp.where(d2 == m, iota, _N), axis=1, keepdims=True)
        idxs.append(am)
        vals.append(m)
        d2 = jnp.where(iota == am, big, d2)
    idx_ref[0] = jnp.concatenate(idxs, axis=1) + b * _N
    dist_ref[0] = jnp.sqrt(jnp.concatenate(vals, axis=1))


def _knn_pass(xyz, xyzT):
    nr = _N // _ROWS
    return pl.pallas_call(
        _knn_kernel,
        grid=(_B, nr),
        in_specs=[
            pl.BlockSpec((1, _ROWS, 3), lambda b, r: (b, r, 0)),
            pl.BlockSpec((1, 3, _N), lambda b, r: (b, 0, 0)),
        ],
        out_specs=[
            pl.BlockSpec((1, _ROWS, _KP1), lambda b, r: (b, r, 0)),
            pl.BlockSpec((1, _ROWS, _KP1), lambda b, r: (b, r, 0)),
        ],
        out_shape=[
            jax.ShapeDtypeStruct((_B, _N, _KP1), jnp.int32),
            jax.ShapeDtypeStruct((_B, _N, _KP1), jnp.float32),
        ],
    )(xyz, xyzT)


def _fuv_kernel(p_ref, xt_ref, wn_ref, wcm_ref, wx_ref, s1_ref, t1_ref,
                u_ref, v_ref):
    p = p_ref[0]
    x = xt_ref[0]
    f = _lrelu(p * s1_ref[...] + t1_ref[...])
    wx = wx_ref[...]
    gx = (wx[:, 0:1] * x[0:1, :] + wx[:, 1:2] * x[1:2, :]
          + wx[:, 2:3] * x[2:3, :])
    u_ref[0] = jnp.dot(wn_ref[...], f, preferred_element_type=jnp.float32) + gx
    v_ref[0] = jnp.dot(wcm_ref[...], f, preferred_element_type=jnp.float32) - gx


def _fuv_pass(P, xyzT, Wn, Wcm, Wx, s1, t1):
    nc = _N // _CB
    return pl.pallas_call(
        _fuv_kernel,
        grid=(_B, nc),
        in_specs=[
            pl.BlockSpec((1, _HID, _CB), lambda b, c: (b, 0, c)),
            pl.BlockSpec((1, 3, _CB), lambda b, c: (b, 0, c)),
            pl.BlockSpec((_COUT, _HID), lambda b, c: (0, 0)),
            pl.BlockSpec((_COUT, _HID), lambda b, c: (0, 0)),
            pl.BlockSpec((_COUT, 3), lambda b, c: (0, 0)),
            pl.BlockSpec((_HID, 1), lambda b, c: (0, 0)),
            pl.BlockSpec((_HID, 1), lambda b, c: (0, 0)),
        ],
        out_specs=[
            pl.BlockSpec((1, _COUT, _CB), lambda b, c: (b, 0, c)),
            pl.BlockSpec((1, _COUT, _CB), lambda b, c: (b, 0, c)),
        ],
        out_shape=[
            jax.ShapeDtypeStruct((_B, _COUT, _N), jnp.float32),
            jax.ShapeDtypeStruct((_B, _COUT, _N), jnp.float32),
        ],
    )(P, xyzT, Wn, Wcm, Wx, s1, t1)


_SC_CHUNK = 512


def _gather_rows(table, idx):
    E = idx.shape[0]
    info = plsc.get_sparse_core_info()
    nw = info.num_cores * info.num_subcores
    per_w = E // nw
    nchunk = per_w // _SC_CHUNK
    mesh = plsc.VectorSubcoreMesh(core_axis_name="c", subcore_axis_name="s")

    @functools.partial(
        pl.kernel,
        out_type=jax.ShapeDtypeStruct((E, _COUT), jnp.float32),
        mesh=mesh,
        scratch_types=[
            pltpu.VMEM((_SC_CHUNK,), jnp.int32),
            pltpu.VMEM((_SC_CHUNK, _COUT), jnp.float32),
            pltpu.SemaphoreType.DMA,
        ],
    )
    def sc_gather(table_hbm, idx_hbm, out_hbm, idx_v, rows_v, sem):
        wid = lax.axis_index("s") * info.num_cores + lax.axis_index("c")
        base = wid * per_w
        for c in range(nchunk):
            off = base + c * _SC_CHUNK
            pltpu.sync_copy(idx_hbm.at[pl.ds(off, _SC_CHUNK)], idx_v)
            pltpu.async_copy(table_hbm.at[idx_v], rows_v, sem).wait()
            pltpu.sync_copy(rows_v, out_hbm.at[pl.ds(off, _SC_CHUNK)])

    return sc_gather(table, idx)


def _edge_e1(g_ref, vt_ref, d_ref, wd_ref):
    u = g_ref[...].reshape(_EPTS, _K, _COUT)
    d = d_ref[...]
    vt = vt_ref[...]
    wd = wd_ref[...]
    return u + vt[:, None, :] + d[:, :, None] * wd[None, :, :]


def _stats1_kernel(g_ref, vt_ref, d_ref, wd_ref, s_ref):
    @pl.when(pl.program_id(0) == 0)
    def _():
        s_ref[...] = jnp.zeros_like(s_ref)

    e1 = _edge_e1(g_ref, vt_ref, d_ref, wd_ref)
    s_ref[...] += jnp.stack(
        [jnp.sum(e1, axis=(0, 1)), jnp.sum(e1 * e1, axis=(0, 1))], axis=0)


def _stats2_kernel(g_ref, vt_ref, d_ref, wd_ref, s2_ref, t2_ref, we2_ref,
                   s_ref):
    @pl.when(pl.program_id(0) == 0)
    def _():
        s_ref[...] = jnp.zeros_like(s_ref)

    e1 = _edge_e1(g_ref, vt_ref, d_ref, wd_ref)
    h1 = _lrelu(e1 * s2_ref[...][None, :, :] + t2_ref[...][None, :, :])
    e2 = jnp.dot(h1.reshape(_EPTS * _K, _COUT).astype(jnp.bfloat16),
                 we2_ref[...].astype(jnp.bfloat16),
                 preferred_element_type=jnp.float32)
    s_ref[...] += jnp.stack(
        [jnp.sum(e2, axis=0), jnp.sum(e2 * e2, axis=0)], axis=0)


def _final_kernel(g_ref, vt_ref, d_ref, wd_ref, s2_ref, t2_ref, we2_ref,
                  s3_ref, t3_ref, scp_ref, s4_ref, t4_ref, o_ref):
    e1 = _edge_e1(g_ref, vt_ref, d_ref, wd_ref)
    h1 = _lrelu(e1 * s2_ref[...][None, :, :] + t2_ref[...][None, :, :])
    e2 = jnp.dot(h1.reshape(_EPTS * _K, _COUT).astype(jnp.bfloat16),
                 we2_ref[...].astype(jnp.bfloat16),
                 preferred_element_type=jnp.float32)
    h2 = _lrelu(e2 * s3_ref[...] + t3_ref[...]).reshape(_EPTS, _K, _COUT)
    mx = jnp.max(h2, axis=1)
    sc = scp_ref[...] * s4_ref[...] + t4_ref[...]
    o_ref[...] = _lrelu(mx + sc)


def _edge_specs(extra):
    ne = (_B * _N) // _EPTS
    specs = [
        pl.BlockSpec((_EPTS * _K, _COUT), lambda i: (i, 0)),
        pl.BlockSpec((_EPTS, _COUT), lambda i: (i, 0)),
        pl.BlockSpec((_EPTS, _K), lambda i: (i, 0)),
        pl.BlockSpec((1, _COUT), lambda i: (0, 0)),
    ]
    specs += [pl.BlockSpec(s, lambda i: (0, 0)) for s in extra]
    return ne, specs


def _stats1_pass(G, vT, dist, wd):
    ne, specs = _edge_specs([])
    return pl.pallas_call(
        _stats1_kernel, grid=(ne,), in_specs=specs,
        out_specs=pl.BlockSpec((2, _COUT), lambda i: (0, 0)),
        out_shape=jax.ShapeDtypeStruct((2, _COUT), jnp.float32),
    )(G, vT, dist, wd)


def _stats2_pass(G, vT, dist, wd, s2, t2, We2T):
    ne, specs = _edge_specs([(1, _COUT), (1, _COUT), (_COUT, _COUT)])
    return pl.pallas_call(
        _stats2_kernel, grid=(ne,), in_specs=specs,
        out_specs=pl.BlockSpec((2, _COUT), lambda i: (0, 0)),
        out_shape=jax.ShapeDtypeStruct((2, _COUT), jnp.float32),
    )(G, vT, dist, wd, s2, t2, We2T)


def _final_pass(G, vT, dist, wd, s2, t2, We2T, s3, t3, SCpT, s4, t4):
    ne, specs = _edge_specs([(1, _COUT), (1, _COUT), (_COUT, _COUT),
                             (1, _COUT), (1, _COUT)])
    specs.append(pl.BlockSpec((_EPTS, _COUT), lambda i: (i, 0)))
    specs += [pl.BlockSpec((1, _COUT), lambda i: (0, 0))] * 2
    return pl.pallas_call(
        _final_kernel, grid=(ne,), in_specs=specs,
        out_specs=pl.BlockSpec((_EPTS, _COUT), lambda i: (i, 0)),
        out_shape=jax.ShapeDtypeStruct((_B * _N, _COUT), jnp.float32),
    )(G, vT, dist, wd, s2, t2, We2T, s3, t3, SCpT, s4, t4)


def _bn_coef(sums, m, g, b):
    mean = sums[:, 0] / m
    var = sums[:, 1] / m - mean * mean
    s = g / jnp.sqrt(var + _EPS)
    return s, b - mean * s


def _bn_coef_row(sums, m, g, b):
    mean = sums[0] / m
    var = sums[1] / m - mean * mean
    s = g / jnp.sqrt(var + _EPS)
    return s.reshape(1, _COUT), (b - mean * s).reshape(1, _COUT)


def kernel(xyz, features, W_pre, g1, b1, W_e1, g2, b2, W_e2, g3, b3,
           W_sc, g4, b4):
    xyzT = jnp.transpose(xyz, (0, 2, 1))

    P, SCp, psums, ssums = _pre_pass(features, W_pre, W_sc)
    s1, t1 = _bn_coef(psums, _B * _N, g1, b1)
    s4r = (g4 / jnp.sqrt(ssums[:, 1] / (_B * _N)
                         - (ssums[:, 0] / (_B * _N)) ** 2 + _EPS))
    t4r = b4 - (ssums[:, 0] / (_B * _N)) * s4r
    s4 = s4r.reshape(1, _COUT)
    t4 = t4r.reshape(1, _COUT)

    idxf, dist = _knn_pass(xyz, xyzT)
    idx_flat = idxf[:, :, 1:].reshape(-1)
    dist16 = dist[:, :, 1:].reshape(_B * _N, _K)

    Wc = W_e1[:, :_HID]
    Wn = W_e1[:, _HID:2 * _HID]
    Wx = W_e1[:, 2 * _HID:2 * _HID + 3]
    wd = W_e1[:, 2 * _HID + 3].reshape(1, _COUT)
    u, v = _fuv_pass(P, xyzT, Wn, Wc - Wn, Wx,
                     s1.reshape(_HID, 1), t1.reshape(_HID, 1))

    uT = u.transpose(0, 2, 1).reshape(_B * _N, _COUT)
    vT = v.transpose(0, 2, 1).reshape(_B * _N, _COUT)
    SCpT = SCp.transpose(0, 2, 1).reshape(_B * _N, _COUT)

    G = _gather_rows(uT, idx_flat)

    M = _B * _N * _K
    sums1 = _stats1_pass(G, vT, dist16, wd)
    s2, t2 = _bn_coef_row(sums1, M, g2, b2)
    sums2 = _stats2_pass(G, vT, dist16, wd, s2, t2, W_e2.T)
    s3, t3 = _bn_coef_row(sums2, M, g3, b3)

    out = _final_pass(G, vT, dist16, wd, s2, t2, W_e2.T, s3, t3, SCpT, s4, t4)
    return out.reshape(_B, _N, _COUT).transpose(0, 2, 1)


# speedup vs baseline: 1.1562x; 1.0955x over previous
import functools

import jax
import jax.numpy as jnp
from jax import lax
from jax.experimental import pallas as pl
from jax.experimental.pallas import tpu as pltpu
from jax.experimental.pallas import tpu_sc as plsc

_B, _N, _CIN, _COUT, _K = 4, 4096, 64, 128, 16
_HID = _COUT // 2
_KP1 = _K + 1
_EPS = 1e-5
_ROWS = 256
_EPTS = 256
_CB = 2048


def _lrelu(x):
    return jnp.where(x >= 0, x, 0.2 * x)


def _pre_kernel(f_ref, wpre_ref, wsc_ref, p_ref, scp_ref, ps_ref, ss_ref):
    b = pl.program_id(0)
    c = pl.program_id(1)

    @pl.when(jnp.logical_and(b == 0, c == 0))
    def _():
        ps_ref[...] = jnp.zeros_like(ps_ref)
        ss_ref[...] = jnp.zeros_like(ss_ref)

    fb = f_ref[0]
    p = jnp.dot(wpre_ref[...], fb, preferred_element_type=jnp.float32)
    sc = jnp.dot(wsc_ref[...], fb, preferred_element_type=jnp.float32)
    p_ref[0] = p
    scp_ref[0] = sc
    ps_ref[...] += jnp.concatenate(
        [jnp.sum(p, axis=1, keepdims=True), jnp.sum(p * p, axis=1, keepdims=True)], axis=1)
    ss_ref[...] += jnp.concatenate(
        [jnp.sum(sc, axis=1, keepdims=True), jnp.sum(sc * sc, axis=1, keepdims=True)], axis=1)


def _pre_pass(features, W_pre, W_sc):
    nc = _N // _CB
    return pl.pallas_call(
        _pre_kernel,
        grid=(_B, nc),
        in_specs=[
            pl.BlockSpec((1, _CIN, _CB), lambda b, c: (b, 0, c)),
            pl.BlockSpec((_HID, _CIN), lambda b, c: (0, 0)),
            pl.BlockSpec((_COUT, _CIN), lambda b, c: (0, 0)),
        ],
        out_specs=[
            pl.BlockSpec((1, _HID, _CB), lambda b, c: (b, 0, c)),
            pl.BlockSpec((1, _COUT, _CB), lambda b, c: (b, 0, c)),
            pl.BlockSpec((_HID, 2), lambda b, c: (0, 0)),
            pl.BlockSpec((_COUT, 2), lambda b, c: (0, 0)),
        ],
        out_shape=[
            jax.ShapeDtypeStruct((_B, _HID, _N), jnp.float32),
            jax.ShapeDtypeStruct((_B, _COUT, _N), jnp.float32),
            jax.ShapeDtypeStruct((_HID, 2), jnp.float32),
            jax.ShapeDtypeStruct((_COUT, 2), jnp.float32),
        ],
    )(features, W_pre, W_sc)


def _knn_kernel(xq_ref, xt_ref, idx_ref, dist_ref):
    b = pl.program_id(0)
    xq = xq_ref[0]
    xt = xt_ref[0]
    sqa = jnp.sum(xt * xt, axis=0, keepdims=True)
    sqq = jnp.sum(xq * xq, axis=1, keepdims=True)
    mm = jnp.dot(xq.astype(jnp.bfloat16), xt.astype(jnp.bfloat16),
                 preferred_element_type=jnp.float32)
    d2 = jnp.maximum(sqq + sqa - 2.0 * mm, 0.0)

    iota = lax.broadcasted_iota(jnp.int32, (_ROWS, _N), 1)
    big = jnp.float32(jnp.inf)
    idxs, vals = [], []
    for _ in range(_KP1):
        m = jnp.min(d2, axis=1, keepdims=True)
        am = jnp.min(jnp.where(d2 == m, iota, _N), axis=1, keepdims=True)
        idxs.append(am)
        vals.append(m)
        d2 = jnp.where(iota == am, big, d2)
    idx_ref[0] = jnp.concatenate(idxs, axis=1) + b * _N
    dist_ref[0] = jnp.sqrt(jnp.concatenate(vals, axis=1))


def _knn_pass(xyz, xyzT):
    nr = _N // _ROWS
    return pl.pallas_call(
        _knn_kernel,
        grid=(_B, nr),
        in_specs=[
            pl.BlockSpec((1, _ROWS, 3), lambda b, r: (b, r, 0)),
            pl.BlockSpec((1, 3, _N), lambda b, r: (b, 0, 0)),
        ],
        out_specs=[
            pl.BlockSpec((1, _ROWS, _KP1), lambda b, r: (b, r, 0)),
            pl.BlockSpec((1, _ROWS, _KP1), lambda b, r: (b, r, 0)),
        ],
        out_shape=[
            jax.ShapeDtypeStruct((_B, _N, _KP1), jnp.int32),
            jax.ShapeDtypeStruct((_B, _N, _KP1), jnp.float32),
        ],
    )(xyz, xyzT)


def _fuv_kernel(p_ref, xt_ref, wn_ref, wcm_ref, wx_ref, s1_ref, t1_ref,
                u_ref, v_ref):
    p = p_ref[0]
    x = xt_ref[0]
    f = _lrelu(p * s1_ref[...] + t1_ref[...])
    wx = wx_ref[...]
    gx = (wx[:, 0:1] * x[0:1, :] + wx[:, 1:2] * x[1:2, :]
          + wx[:, 2:3] * x[2:3, :])
    u_ref[0] = jnp.dot(wn_ref[...], f, preferred_element_type=jnp.float32) + gx
    v_ref[0] = jnp.dot(wcm_ref[...], f, preferred_element_type=jnp.float32) - gx


def _fuv_pass(P, xyzT, Wn, Wcm, Wx, s1, t1):
    nc = _N // _CB
    return pl.pallas_call(
        _fuv_kernel,
        grid=(_B, nc),
        in_specs=[
            pl.BlockSpec((1, _HID, _CB), lambda b, c: (b, 0, c)),
            pl.BlockSpec((1, 3, _CB), lambda b, c: (b, 0, c)),
            pl.BlockSpec((_COUT, _HID), lambda b, c: (0, 0)),
            pl.BlockSpec((_COUT, _HID), lambda b, c: (0, 0)),
            pl.BlockSpec((_COUT, 3), lambda b, c: (0, 0)),
            pl.BlockSpec((_HID, 1), lambda b, c: (0, 0)),
            pl.BlockSpec((_HID, 1), lambda b, c: (0, 0)),
        ],
        out_specs=[
            pl.BlockSpec((1, _COUT, _CB), lambda b, c: (b, 0, c)),
            pl.BlockSpec((1, _COUT, _CB), lambda b, c: (b, 0, c)),
        ],
        out_shape=[
            jax.ShapeDtypeStruct((_B, _COUT, _N), jnp.float32),
            jax.ShapeDtypeStruct((_B, _COUT, _N), jnp.float32),
        ],
    )(P, xyzT, Wn, Wcm, Wx, s1, t1)


_SC_CHUNK = 512


def _gather_rows(table, idx):
    E = idx.shape[0]
    info = plsc.get_sparse_core_info()
    nw = info.num_cores * info.num_subcores
    per_w = E // nw
    nchunk = per_w // _SC_CHUNK
    mesh = plsc.VectorSubcoreMesh(core_axis_name="c", subcore_axis_name="s")

    @functools.partial(
        pl.kernel,
        out_type=jax.ShapeDtypeStruct((E, _COUT), jnp.float32),
        mesh=mesh,
        scratch_types=[
            pltpu.VMEM((_SC_CHUNK,), jnp.int32),
            pltpu.VMEM((_SC_CHUNK, _COUT), jnp.float32),
            pltpu.SemaphoreType.DMA,
        ],
    )
    def sc_gather(table_hbm, idx_hbm, out_hbm, idx_v, rows_v, sem):
        wid = lax.axis_index("s") * info.num_cores + lax.axis_index("c")
        base = wid * per_w
        for c in range(nchunk):
            off = base + c * _SC_CHUNK
            pltpu.sync_copy(idx_hbm.at[pl.ds(off, _SC_CHUNK)], idx_v)
            pltpu.async_copy(table_hbm.at[idx_v], rows_v, sem).wait()
            pltpu.sync_copy(rows_v, out_hbm.at[pl.ds(off, _SC_CHUNK)])

    return sc_gather(table, idx)


def _edge_e1(g_ref, vt_ref, d_ref, wd_ref):
    u = g_ref[...].reshape(_EPTS, _K, _COUT)
    d = d_ref[...]
    vt = vt_ref[...]
    wd = wd_ref[...]
    return u + vt[:, None, :] + d[:, :, None] * wd[None, :, :]


def _stats1_kernel(g_ref, vt_ref, d_ref, wd_ref, s_ref):
    @pl.when(pl.program_id(0) == 0)
    def _():
        s_ref[...] = jnp.zeros_like(s_ref)

    e1 = _edge_e1(g_ref, vt_ref, d_ref, wd_ref)
    s_ref[...] += jnp.stack(
        [jnp.sum(e1, axis=(0, 1)), jnp.sum(e1 * e1, axis=(0, 1))], axis=0)


def _stats2_kernel(g_ref, vt_ref, d_ref, wd_ref, s2_ref, t2_ref, we2_ref,
                   s_ref):
    @pl.when(pl.program_id(0) == 0)
    def _():
        s_ref[...] = jnp.zeros_like(s_ref)

    e1 = _edge_e1(g_ref, vt_ref, d_ref, wd_ref)
    h1 = _lrelu(e1 * s2_ref[...][None, :, :] + t2_ref[...][None, :, :])
    e2 = jnp.dot(h1.reshape(_EPTS * _K, _COUT).astype(jnp.bfloat16),
                 we2_ref[...].astype(jnp.bfloat16),
                 preferred_element_type=jnp.float32)
    s_ref[...] += jnp.stack(
        [jnp.sum(e2, axis=0), jnp.sum(e2 * e2, axis=0)], axis=0)


def _final_kernel(g_ref, vt_ref, d_ref, wd_ref, s2_ref, t2_ref, we2_ref,
                  s3_ref, t3_ref, scp_ref, s4_ref, t4_ref, o_ref):
    e1 = _edge_e1(g_ref, vt_ref, d_ref, wd_ref)
    h1 = _lrelu(e1 * s2_ref[...][None, :, :] + t2_ref[...][None, :, :])
    e2 = jnp.dot(h1.reshape(_EPTS * _K, _COUT).astype(jnp.bfloat16),
                 we2_ref[...].astype(jnp.bfloat16),
                 preferred_element_type=jnp.float32)
    h2 = _lrelu(e2 * s3_ref[...] + t3_ref[...]).reshape(_EPTS, _K, _COUT)
    mx = jnp.max(h2, axis=1)
    sc = scp_ref[...] * s4_ref[...] + t4_ref[...]
    o_ref[...] = _lrelu(mx + sc)


def _edge_specs(extra):
    ne = (_B * _N) // _EPTS
    specs = [
        pl.BlockSpec((_EPTS * _K, _COUT), lambda i: (i, 0)),
        pl.BlockSpec((_EPTS, _COUT), lambda i: (i, 0)),
        pl.BlockSpec((_EPTS, _K), lambda i: (i, 0)),
        pl.BlockSpec((1, _COUT), lambda i: (0, 0)),
    ]
    specs += [pl.BlockSpec(s, lambda i: (0, 0)) for s in extra]
    return ne, specs


def _stats1_pass(G, vT, dist, wd):
    ne, specs = _edge_specs([])
    return pl.pallas_call(
        _stats1_kernel, grid=(ne,), in_specs=specs,
        out_specs=pl.BlockSpec((2, _COUT), lambda i: (0, 0)),
        out_shape=jax.ShapeDtypeStruct((2, _COUT), jnp.float32),
    )(G, vT, dist, wd)


def _stats2_pass(G, vT, dist, wd, s2, t2, We2T):
    ne, specs = _edge_specs([(1, _COUT), (1, _COUT), (_COUT, _COUT)])
    return pl.pallas_call(
        _stats2_kernel, grid=(ne,), in_specs=specs,
        out_specs=pl.BlockSpec((2, _COUT), lambda i: (0, 0)),
        out_shape=jax.ShapeDtypeStruct((2, _COUT), jnp.float32),
    )(G, vT, dist, wd, s2, t2, We2T)


def _final_pass(G, vT, dist, wd, s2, t2, We2T, s3, t3, SCpT, s4, t4):
    ne, specs = _edge_specs([(1, _COUT), (1, _COUT), (_COUT, _COUT),
                             (1, _COUT), (1, _COUT)])
    specs.append(pl.BlockSpec((_EPTS, _COUT), lambda i: (i, 0)))
    specs += [pl.BlockSpec((1, _COUT), lambda i: (0, 0))] * 2
    return pl.pallas_call(
        _final_kernel, grid=(ne,), in_specs=specs,
        out_specs=pl.BlockSpec((_EPTS, _COUT), lambda i: (i, 0)),
        out_shape=jax.ShapeDtypeStruct((_B * _N, _COUT), jnp.float32),
    )(G, vT, dist, wd, s2, t2, We2T, s3, t3, SCpT, s4, t4)


def _bn_coef(sums, m, g, b):
    mean = sums[:, 0] / m
    var = sums[:, 1] / m - mean * mean
    s = g / jnp.sqrt(var + _EPS)
    return s, b - mean * s


def _bn_coef_row(sums, m, g, b):
    mean = sums[0] / m
    var = sums[1] / m - mean * mean
    s = g / jnp.sqrt(var + _EPS)
    return s.reshape(1, _COUT), (b - mean * s).reshape(1, _COUT)


def kernel(xyz, features, W_pre, g1, b1, W_e1, g2, b2, W_e2, g3, b3,
           W_sc, g4, b4):
    xyzT = jnp.transpose(xyz, (0, 2, 1))

    P, SCp, psums, ssums = _pre_pass(features, W_pre, W_sc)
    s1, t1 = _bn_coef(psums, _B * _N, g1, b1)
    s4r = (g4 / jnp.sqrt(ssums[:, 1] / (_B * _N)
                         - (ssums[:, 0] / (_B * _N)) ** 2 + _EPS))
    t4r = b4 - (ssums[:, 0] / (_B * _N)) * s4r
    s4 = s4r.reshape(1, _COUT)
    t4 = t4r.reshape(1, _COUT)

    idxf, dist = _knn_pass(xyz, xyzT)
    idx_flat = idxf[:, :, 1:].reshape(-1)
    dist16 = dist[:, :, 1:].reshape(_B * _N, _K)

    Wc = W_e1[:, :_HID]
    Wn = W_e1[:, _HID:2 * _HID]
    Wx = W_e1[:, 2 * _HID:2 * _HID + 3]
    wd = W_e1[:, 2 * _HID + 3].reshape(1, _COUT)
    u, v = _fuv_pass(P, xyzT, Wn, Wc - Wn, Wx,
                     s1.reshape(_HID, 1), t1.reshape(_HID, 1))

    uT = u.transpose(0, 2, 1).reshape(_B * _N, _COUT)
    vT = v.transpose(0, 2, 1).reshape(_B * _N, _COUT)
    SCpT = SCp.transpose(0, 2, 1).reshape(_B * _N, _COUT)

    G = _gather_rows(uT, idx_flat)

    M = _B * _N * _K
    sums1 = _stats1_pass(G, vT, dist16, wd)
    s2, t2 = _bn_coef_row(sums1, M, g2, b2)
    sums2 = _stats2_pass(G, vT, dist16, wd, s2, t2, W_e2.T)
    s3, t3 = _bn_coef_row(sums2, M, g3, b3)

    out = _final_pass(G, vT, dist16, wd, s2, t2, W_e2.T, s3, t3, SCpT, s4, t4)
    return out.reshape(_B, _N, _COUT).transpose(0, 2, 1)


# speedup vs baseline: 1.1852x; 1.0251x over previous
import functools

import jax
import jax.numpy as jnp
from jax import lax
from jax.experimental import pallas as pl
from jax.experimental.pallas import tpu as pltpu
from jax.experimental.pallas import tpu_sc as plsc

_B, _N, _CIN, _COUT, _K = 4, 4096, 64, 128, 16
_HID = _COUT // 2
_KP1 = _K + 1
_EPS = 1e-5
_ROWS = 256
_EPTS = 512
_CB = 2048


def _lrelu(x):
    return jnp.where(x >= 0, x, 0.2 * x)


def _pre_kernel(f_ref, wpre_ref, wsc_ref, p_ref, scp_ref, ps_ref, ss_ref):
    b = pl.program_id(0)
    c = pl.program_id(1)

    @pl.when(jnp.logical_and(b == 0, c == 0))
    def _():
        ps_ref[...] = jnp.zeros_like(ps_ref)
        ss_ref[...] = jnp.zeros_like(ss_ref)

    fb = f_ref[0]
    p = jnp.dot(wpre_ref[...], fb, preferred_element_type=jnp.float32)
    sc = jnp.dot(wsc_ref[...], fb, preferred_element_type=jnp.float32)
    p_ref[0] = p
    scp_ref[0] = sc
    ps_ref[...] += jnp.concatenate(
        [jnp.sum(p, axis=1, keepdims=True), jnp.sum(p * p, axis=1, keepdims=True)], axis=1)
    ss_ref[...] += jnp.concatenate(
        [jnp.sum(sc, axis=1, keepdims=True), jnp.sum(sc * sc, axis=1, keepdims=True)], axis=1)


def _pre_pass(features, W_pre, W_sc):
    nc = _N // _CB
    return pl.pallas_call(
        _pre_kernel,
        grid=(_B, nc),
        in_specs=[
            pl.BlockSpec((1, _CIN, _CB), lambda b, c: (b, 0, c)),
            pl.BlockSpec((_HID, _CIN), lambda b, c: (0, 0)),
            pl.BlockSpec((_COUT, _CIN), lambda b, c: (0, 0)),
        ],
        out_specs=[
            pl.BlockSpec((1, _HID, _CB), lambda b, c: (b, 0, c)),
            pl.BlockSpec((1, _COUT, _CB), lambda b, c: (b, 0, c)),
            pl.BlockSpec((_HID, 2), lambda b, c: (0, 0)),
            pl.BlockSpec((_COUT, 2), lambda b, c: (0, 0)),
        ],
        out_shape=[
            jax.ShapeDtypeStruct((_B, _HID, _N), jnp.float32),
            jax.ShapeDtypeStruct((_B, _COUT, _N), jnp.float32),
            jax.ShapeDtypeStruct((_HID, 2), jnp.float32),
            jax.ShapeDtypeStruct((_COUT, 2), jnp.float32),
        ],
    )(features, W_pre, W_sc)


def _knn_kernel(xq_ref, xt_ref, idx_ref, dist_ref):
    b = pl.program_id(0)
    xq = xq_ref[0]
    xt = xt_ref[0]
    sqa = jnp.sum(xt * xt, axis=0, keepdims=True)
    sqq = jnp.sum(xq * xq, axis=1, keepdims=True)
    mm = jnp.dot(xq.astype(jnp.bfloat16), xt.astype(jnp.bfloat16),
                 preferred_element_type=jnp.float32)
    d2 = jnp.maximum(sqq + sqa - 2.0 * mm, 0.0)

    iota = lax.broadcasted_iota(jnp.int32, (_ROWS, _N), 1)
    big = jnp.float32(jnp.inf)
    idxs, vals = [], []
    for _ in range(_KP1):
        m = jnp.min(d2, axis=1, keepdims=True)
        am = jnp.min(jnp.where(d2 == m, iota, _N), axis=1, keepdims=True)
        idxs.append(am)
        vals.append(m)
        d2 = jnp.where(iota == am, big, d2)
    idx_ref[0] = jnp.concatenate(idxs, axis=1) + b * _N
    dist_ref[0] = jnp.sqrt(jnp.concatenate(vals, axis=1))


def _knn_pass(xyz, xyzT):
    nr = _N // _ROWS
    return pl.pallas_call(
        _knn_kernel,
        grid=(_B, nr),
        in_specs=[
            pl.BlockSpec((1, _ROWS, 3), lambda b, r: (b, r, 0)),
            pl.BlockSpec((1, 3, _N), lambda b, r: (b, 0, 0)),
        ],
        out_specs=[
            pl.BlockSpec((1, _ROWS, _KP1), lambda b, r: (b, r, 0)),
            pl.BlockSpec((1, _ROWS, _KP1), lambda b, r: (b, r, 0)),
        ],
        out_shape=[
            jax.ShapeDtypeStruct((_B, _N, _KP1), jnp.int32),
            jax.ShapeDtypeStruct((_B, _N, _KP1), jnp.float32),
        ],
    )(xyz, xyzT)


def _fuv_kernel(p_ref, xt_ref, wn_ref, wcm_ref, wx_ref, s1_ref, t1_ref,
                u_ref, v_ref):
    p = p_ref[0]
    x = xt_ref[0]
    f = _lrelu(p * s1_ref[...] + t1_ref[...])
    wx = wx_ref[...]
    gx = (wx[:, 0:1] * x[0:1, :] + wx[:, 1:2] * x[1:2, :]
          + wx[:, 2:3] * x[2:3, :])
    u_ref[0] = jnp.dot(wn_ref[...], f, preferred_element_type=jnp.float32) + gx
    v_ref[0] = jnp.dot(wcm_ref[...], f, preferred_element_type=jnp.float32) - gx


def _fuv_pass(P, xyzT, Wn, Wcm, Wx, s1, t1):
    nc = _N // _CB
    return pl.pallas_call(
        _fuv_kernel,
        grid=(_B, nc),
        in_specs=[
            pl.BlockSpec((1, _HID, _CB), lambda b, c: (b, 0, c)),
            pl.BlockSpec((1, 3, _CB), lambda b, c: (b, 0, c)),
            pl.BlockSpec((_COUT, _HID), lambda b, c: (0, 0)),
            pl.BlockSpec((_COUT, _HID), lambda b, c: (0, 0)),
            pl.BlockSpec((_COUT, 3), lambda b, c: (0, 0)),
            pl.BlockSpec((_HID, 1), lambda b, c: (0, 0)),
            pl.BlockSpec((_HID, 1), lambda b, c: (0, 0)),
        ],
        out_specs=[
            pl.BlockSpec((1, _COUT, _CB), lambda b, c: (b, 0, c)),
            pl.BlockSpec((1, _COUT, _CB), lambda b, c: (b, 0, c)),
        ],
        out_shape=[
            jax.ShapeDtypeStruct((_B, _COUT, _N), jnp.float32),
            jax.ShapeDtypeStruct((_B, _COUT, _N), jnp.float32),
        ],
    )(P, xyzT, Wn, Wcm, Wx, s1, t1)


_SC_CHUNK = 512


def _gather_rows(table, idx):
    E = idx.shape[0]
    info = plsc.get_sparse_core_info()
    nw = info.num_cores * info.num_subcores
    per_w = E // nw
    nchunk = per_w // _SC_CHUNK
    mesh = plsc.VectorSubcoreMesh(core_axis_name="c", subcore_axis_name="s")

    @functools.partial(
        pl.kernel,
        out_type=jax.ShapeDtypeStruct((E, _COUT), jnp.float32),
        mesh=mesh,
        scratch_types=[
            pltpu.VMEM((_SC_CHUNK,), jnp.int32),
            pltpu.VMEM((_SC_CHUNK, _COUT), jnp.float32),
            pltpu.SemaphoreType.DMA,
        ],
    )
    def sc_gather(table_hbm, idx_hbm, out_hbm, idx_v, rows_v, sem):
        wid = lax.axis_index("s") * info.num_cores + lax.axis_index("c")
        base = wid * per_w
        for c in range(nchunk):
            off = base + c * _SC_CHUNK
            pltpu.sync_copy(idx_hbm.at[pl.ds(off, _SC_CHUNK)], idx_v)
            pltpu.async_copy(table_hbm.at[idx_v], rows_v, sem).wait()
            pltpu.sync_copy(rows_v, out_hbm.at[pl.ds(off, _SC_CHUNK)])

    return sc_gather(table, idx)


def _edge_e1(g_ref, vt_ref, d_ref, wd_ref):
    u = g_ref[...].reshape(_EPTS, _K, _COUT)
    d = d_ref[...]
    vt = vt_ref[...]
    wd = wd_ref[...]
    return u + vt[:, None, :] + d[:, :, None] * wd[None, :, :]


def _stats1_kernel(g_ref, vt_ref, d_ref, wd_ref, s_ref):
    @pl.when(pl.program_id(0) == 0)
    def _():
        s_ref[...] = jnp.zeros_like(s_ref)

    e1 = _edge_e1(g_ref, vt_ref, d_ref, wd_ref)
    s_ref[...] += jnp.stack(
        [jnp.sum(e1, axis=(0, 1)), jnp.sum(e1 * e1, axis=(0, 1))], axis=0)


def _stats2_kernel(g_ref, vt_ref, d_ref, wd_ref, s2_ref, t2_ref, we2_ref,
                   s_ref):
    @pl.when(pl.program_id(0) == 0)
    def _():
        s_ref[...] = jnp.zeros_like(s_ref)

    e1 = _edge_e1(g_ref, vt_ref, d_ref, wd_ref)
    h1 = _lrelu(e1 * s2_ref[...][None, :, :] + t2_ref[...][None, :, :])
    e2 = jnp.dot(h1.reshape(_EPTS * _K, _COUT).astype(jnp.bfloat16),
                 we2_ref[...].astype(jnp.bfloat16),
                 preferred_element_type=jnp.float32)
    s_ref[...] += jnp.stack(
        [jnp.sum(e2, axis=0), jnp.sum(e2 * e2, axis=0)], axis=0)


def _final_kernel(g_ref, vt_ref, d_ref, wd_ref, s2_ref, t2_ref, we2_ref,
                  s3_ref, t3_ref, scp_ref, s4_ref, t4_ref, o_ref):
    e1 = _edge_e1(g_ref, vt_ref, d_ref, wd_ref)
    h1 = _lrelu(e1 * s2_ref[...][None, :, :] + t2_ref[...][None, :, :])
    e2 = jnp.dot(h1.reshape(_EPTS * _K, _COUT).astype(jnp.bfloat16),
                 we2_ref[...].astype(jnp.bfloat16),
                 preferred_element_type=jnp.float32)
    h2 = _lrelu(e2 * s3_ref[...] + t3_ref[...]).reshape(_EPTS, _K, _COUT)
    mx = jnp.max(h2, axis=1)
    sc = scp_ref[...] * s4_ref[...] + t4_ref[...]
    o_ref[...] = _lrelu(mx + sc)


def _edge_specs(extra):
    ne = (_B * _N) // _EPTS
    specs = [
        pl.BlockSpec((_EPTS * _K, _COUT), lambda i: (i, 0)),
        pl.BlockSpec((_EPTS, _COUT), lambda i: (i, 0)),
        pl.BlockSpec((_EPTS, _K), lambda i: (i, 0)),
        pl.BlockSpec((1, _COUT), lambda i: (0, 0)),
    ]
    specs += [pl.BlockSpec(s, lambda i: (0, 0)) for s in extra]
    return ne, specs


def _stats1_pass(G, vT, dist, wd):
    ne, specs = _edge_specs([])
    return pl.pallas_call(
        _stats1_kernel, grid=(ne,), in_specs=specs,
        out_specs=pl.BlockSpec((2, _COUT), lambda i: (0, 0)),
        out_shape=jax.ShapeDtypeStruct((2, _COUT), jnp.float32),
    )(G, vT, dist, wd)


def _stats2_pass(G, vT, dist, wd, s2, t2, We2T):
    ne, specs = _edge_specs([(1, _COUT), (1, _COUT), (_COUT, _COUT)])
    return pl.pallas_call(
        _stats2_kernel, grid=(ne,), in_specs=specs,
        out_specs=pl.BlockSpec((2, _COUT), lambda i: (0, 0)),
        out_shape=jax.ShapeDtypeStruct((2, _COUT), jnp.float32),
    )(G, vT, dist, wd, s2, t2, We2T)


def _final_pass(G, vT, dist, wd, s2, t2, We2T, s3, t3, SCpT, s4, t4):
    ne, specs = _edge_specs([(1, _COUT), (1, _COUT), (_COUT, _COUT),
                             (1, _COUT), (1, _COUT)])
    specs.append(pl.BlockSpec((_EPTS, _COUT), lambda i: (i, 0)))
    specs += [pl.BlockSpec((1, _COUT), lambda i: (0, 0))] * 2
    return pl.pallas_call(
        _final_kernel, grid=(ne,), in_specs=specs,
        out_specs=pl.BlockSpec((_EPTS, _COUT), lambda i: (i, 0)),
        out_shape=jax.ShapeDtypeStruct((_B * _N, _COUT), jnp.float32),
    )(G, vT, dist, wd, s2, t2, We2T, s3, t3, SCpT, s4, t4)


def _bn_coef(sums, m, g, b):
    mean = sums[:, 0] / m
    var = sums[:, 1] / m - mean * mean
    s = g / jnp.sqrt(var + _EPS)
    return s, b - mean * s


def _bn_coef_row(sums, m, g, b):
    mean = sums[0] / m
    var = sums[1] / m - mean * mean
    s = g / jnp.sqrt(var + _EPS)
    return s.reshape(1, _COUT), (b - mean * s).reshape(1, _COUT)


def kernel(xyz, features, W_pre, g1, b1, W_e1, g2, b2, W_e2, g3, b3,
           W_sc, g4, b4):
    xyzT = jnp.transpose(xyz, (0, 2, 1))

    P, SCp, psums, ssums = _pre_pass(features, W_pre, W_sc)
    s1, t1 = _bn_coef(psums, _B * _N, g1, b1)
    s4r = (g4 / jnp.sqrt(ssums[:, 1] / (_B * _N)
                         - (ssums[:, 0] / (_B * _N)) ** 2 + _EPS))
    t4r = b4 - (ssums[:, 0] / (_B * _N)) * s4r
    s4 = s4r.reshape(1, _COUT)
    t4 = t4r.reshape(1, _COUT)

    idxf, dist = _knn_pass(xyz, xyzT)
    idx_flat = idxf[:, :, 1:].reshape(-1)
    dist16 = dist[:, :, 1:].reshape(_B * _N, _K)

    Wc = W_e1[:, :_HID]
    Wn = W_e1[:, _HID:2 * _HID]
    Wx = W_e1[:, 2 * _HID:2 * _HID + 3]
    wd = W_e1[:, 2 * _HID + 3].reshape(1, _COUT)
    u, v = _fuv_pass(P, xyzT, Wn, Wc - Wn, Wx,
                     s1.reshape(_HID, 1), t1.reshape(_HID, 1))

    uT = u.transpose(0, 2, 1).reshape(_B * _N, _COUT)
    vT = v.transpose(0, 2, 1).reshape(_B * _N, _COUT)
    SCpT = SCp.transpose(0, 2, 1).reshape(_B * _N, _COUT)

    G = _gather_rows(uT, idx_flat)

    M = _B * _N * _K
    sums1 = _stats1_pass(G, vT, dist16, wd)
    s2, t2 = _bn_coef_row(sums1, M, g2, b2)
    sums2 = _stats2_pass(G, vT, dist16, wd, s2, t2, W_e2.T)
    s3, t3 = _bn_coef_row(sums2, M, g3, b3)

    out = _final_pass(G, vT, dist16, wd, s2, t2, W_e2.T, s3, t3, SCpT, s4, t4)
    return out.reshape(_B, _N, _COUT).transpose(0, 2, 1)


# speedup vs baseline: 1.1945x; 1.0078x over previous
import functools

import jax
import jax.numpy as jnp
from jax import lax
from jax.experimental import pallas as pl
from jax.experimental.pallas import tpu as pltpu
from jax.experimental.pallas import tpu_sc as plsc

_B, _N, _CIN, _COUT, _K = 4, 4096, 64, 128, 16
_HID = _COUT // 2
_KP1 = _K + 1
_EPS = 1e-5
_ROWS = 256
_EPTS = 1024
_CB = 2048


def _lrelu(x):
    return jnp.where(x >= 0, x, 0.2 * x)


def _pre_kernel(f_ref, wpre_ref, wsc_ref, p_ref, scp_ref, ps_ref, ss_ref):
    b = pl.program_id(0)
    c = pl.program_id(1)

    @pl.when(jnp.logical_and(b == 0, c == 0))
    def _():
        ps_ref[...] = jnp.zeros_like(ps_ref)
        ss_ref[...] = jnp.zeros_like(ss_ref)

    fb = f_ref[0]
    p = jnp.dot(wpre_ref[...], fb, preferred_element_type=jnp.float32)
    sc = jnp.dot(wsc_ref[...], fb, preferred_element_type=jnp.float32)
    p_ref[0] = p
    scp_ref[0] = sc
    ps_ref[...] += jnp.concatenate(
        [jnp.sum(p, axis=1, keepdims=True), jnp.sum(p * p, axis=1, keepdims=True)], axis=1)
    ss_ref[...] += jnp.concatenate(
        [jnp.sum(sc, axis=1, keepdims=True), jnp.sum(sc * sc, axis=1, keepdims=True)], axis=1)


def _pre_pass(features, W_pre, W_sc):
    nc = _N // _CB
    return pl.pallas_call(
        _pre_kernel,
        grid=(_B, nc),
        in_specs=[
            pl.BlockSpec((1, _CIN, _CB), lambda b, c: (b, 0, c)),
            pl.BlockSpec((_HID, _CIN), lambda b, c: (0, 0)),
            pl.BlockSpec((_COUT, _CIN), lambda b, c: (0, 0)),
        ],
        out_specs=[
            pl.BlockSpec((1, _HID, _CB), lambda b, c: (b, 0, c)),
            pl.BlockSpec((1, _COUT, _CB), lambda b, c: (b, 0, c)),
            pl.BlockSpec((_HID, 2), lambda b, c: (0, 0)),
            pl.BlockSpec((_COUT, 2), lambda b, c: (0, 0)),
        ],
        out_shape=[
            jax.ShapeDtypeStruct((_B, _HID, _N), jnp.float32),
            jax.ShapeDtypeStruct((_B, _COUT, _N), jnp.float32),
            jax.ShapeDtypeStruct((_HID, 2), jnp.float32),
            jax.ShapeDtypeStruct((_COUT, 2), jnp.float32),
        ],
    )(features, W_pre, W_sc)


def _knn_kernel(xq_ref, xt_ref, idx_ref, dist_ref):
    b = pl.program_id(0)
    xq = xq_ref[0]
    xt = xt_ref[0]
    sqa = jnp.sum(xt * xt, axis=0, keepdims=True)
    sqq = jnp.sum(xq * xq, axis=1, keepdims=True)
    mm = jnp.dot(xq.astype(jnp.bfloat16), xt.astype(jnp.bfloat16),
                 preferred_element_type=jnp.float32)
    d2 = jnp.maximum(sqq + sqa - 2.0 * mm, 0.0)

    iota = lax.broadcasted_iota(jnp.int32, (_ROWS, _N), 1)
    big = jnp.float32(jnp.inf)
    idxs, vals = [], []
    for _ in range(_KP1):
        m = jnp.min(d2, axis=1, keepdims=True)
        am = jnp.min(jnp.where(d2 == m, iota, _N), axis=1, keepdims=True)
        idxs.append(am)
        vals.append(m)
        d2 = jnp.where(iota == am, big, d2)
    idx_ref[0] = jnp.concatenate(idxs, axis=1) + b * _N
    dist_ref[0] = jnp.sqrt(jnp.concatenate(vals, axis=1))


def _knn_pass(xyz, xyzT):
    nr = _N // _ROWS
    return pl.pallas_call(
        _knn_kernel,
        grid=(_B, nr),
        in_specs=[
            pl.BlockSpec((1, _ROWS, 3), lambda b, r: (b, r, 0)),
            pl.BlockSpec((1, 3, _N), lambda b, r: (b, 0, 0)),
        ],
        out_specs=[
            pl.BlockSpec((1, _ROWS, _KP1), lambda b, r: (b, r, 0)),
            pl.BlockSpec((1, _ROWS, _KP1), lambda b, r: (b, r, 0)),
        ],
        out_shape=[
            jax.ShapeDtypeStruct((_B, _N, _KP1), jnp.int32),
            jax.ShapeDtypeStruct((_B, _N, _KP1), jnp.float32),
        ],
    )(xyz, xyzT)


def _fuv_kernel(p_ref, xt_ref, wn_ref, wcm_ref, wx_ref, s1_ref, t1_ref,
                u_ref, v_ref):
    p = p_ref[0]
    x = xt_ref[0]
    f = _lrelu(p * s1_ref[...] + t1_ref[...])
    wx = wx_ref[...]
    gx = (wx[:, 0:1] * x[0:1, :] + wx[:, 1:2] * x[1:2, :]
          + wx[:, 2:3] * x[2:3, :])
    u_ref[0] = jnp.dot(wn_ref[...], f, preferred_element_type=jnp.float32) + gx
    v_ref[0] = jnp.dot(wcm_ref[...], f, preferred_element_type=jnp.float32) - gx


def _fuv_pass(P, xyzT, Wn, Wcm, Wx, s1, t1):
    nc = _N // _CB
    return pl.pallas_call(
        _fuv_kernel,
        grid=(_B, nc),
        in_specs=[
            pl.BlockSpec((1, _HID, _CB), lambda b, c: (b, 0, c)),
            pl.BlockSpec((1, 3, _CB), lambda b, c: (b, 0, c)),
            pl.BlockSpec((_COUT, _HID), lambda b, c: (0, 0)),
            pl.BlockSpec((_COUT, _HID), lambda b, c: (0, 0)),
            pl.BlockSpec((_COUT, 3), lambda b, c: (0, 0)),
            pl.BlockSpec((_HID, 1), lambda b, c: (0, 0)),
            pl.BlockSpec((_HID, 1), lambda b, c: (0, 0)),
        ],
        out_specs=[
            pl.BlockSpec((1, _COUT, _CB), lambda b, c: (b, 0, c)),
            pl.BlockSpec((1, _COUT, _CB), lambda b, c: (b, 0, c)),
        ],
        out_shape=[
            jax.ShapeDtypeStruct((_B, _COUT, _N), jnp.float32),
            jax.ShapeDtypeStruct((_B, _COUT, _N), jnp.float32),
        ],
    )(P, xyzT, Wn, Wcm, Wx, s1, t1)


_SC_CHUNK = 512


def _gather_rows(table, idx):
    E = idx.shape[0]
    info = plsc.get_sparse_core_info()
    nw = info.num_cores * info.num_subcores
    per_w = E // nw
    nchunk = per_w // _SC_CHUNK
    mesh = plsc.VectorSubcoreMesh(core_axis_name="c", subcore_axis_name="s")

    @functools.partial(
        pl.kernel,
        out_type=jax.ShapeDtypeStruct((E, _COUT), jnp.float32),
        mesh=mesh,
        scratch_types=[
            pltpu.VMEM((_SC_CHUNK,), jnp.int32),
            pltpu.VMEM((_SC_CHUNK, _COUT), jnp.float32),
            pltpu.SemaphoreType.DMA,
        ],
    )
    def sc_gather(table_hbm, idx_hbm, out_hbm, idx_v, rows_v, sem):
        wid = lax.axis_index("s") * info.num_cores + lax.axis_index("c")
        base = wid * per_w
        for c in range(nchunk):
            off = base + c * _SC_CHUNK
            pltpu.sync_copy(idx_hbm.at[pl.ds(off, _SC_CHUNK)], idx_v)
            pltpu.async_copy(table_hbm.at[idx_v], rows_v, sem).wait()
            pltpu.sync_copy(rows_v, out_hbm.at[pl.ds(off, _SC_CHUNK)])

    return sc_gather(table, idx)


def _edge_e1(g_ref, vt_ref, d_ref, wd_ref):
    u = g_ref[...].reshape(_EPTS, _K, _COUT)
    d = d_ref[...]
    vt = vt_ref[...]
    wd = wd_ref[...]
    return u + vt[:, None, :] + d[:, :, None] * wd[None, :, :]


def _stats1_kernel(g_ref, vt_ref, d_ref, wd_ref, s_ref):
    @pl.when(pl.program_id(0) == 0)
    def _():
        s_ref[...] = jnp.zeros_like(s_ref)

    e1 = _edge_e1(g_ref, vt_ref, d_ref, wd_ref)
    s_ref[...] += jnp.stack(
        [jnp.sum(e1, axis=(0, 1)), jnp.sum(e1 * e1, axis=(0, 1))], axis=0)


def _stats2_kernel(g_ref, vt_ref, d_ref, wd_ref, s2_ref, t2_ref, we2_ref,
                   s_ref):
    @pl.when(pl.program_id(0) == 0)
    def _():
        s_ref[...] = jnp.zeros_like(s_ref)

    e1 = _edge_e1(g_ref, vt_ref, d_ref, wd_ref)
    h1 = _lrelu(e1 * s2_ref[...][None, :, :] + t2_ref[...][None, :, :])
    e2 = jnp.dot(h1.reshape(_EPTS * _K, _COUT).astype(jnp.bfloat16),
                 we2_ref[...].astype(jnp.bfloat16),
                 preferred_element_type=jnp.float32)
    s_ref[...] += jnp.stack(
        [jnp.sum(e2, axis=0), jnp.sum(e2 * e2, axis=0)], axis=0)


def _final_kernel(g_ref, vt_ref, d_ref, wd_ref, s2_ref, t2_ref, we2_ref,
                  s3_ref, t3_ref, scp_ref, s4_ref, t4_ref, o_ref):
    e1 = _edge_e1(g_ref, vt_ref, d_ref, wd_ref)
    h1 = _lrelu(e1 * s2_ref[...][None, :, :] + t2_ref[...][None, :, :])
    e2 = jnp.dot(h1.reshape(_EPTS * _K, _COUT).astype(jnp.bfloat16),
                 we2_ref[...].astype(jnp.bfloat16),
                 preferred_element_type=jnp.float32)
    h2 = _lrelu(e2 * s3_ref[...] + t3_ref[...]).reshape(_EPTS, _K, _COUT)
    mx = jnp.max(h2, axis=1)
    sc = scp_ref[...] * s4_ref[...] + t4_ref[...]
    o_ref[...] = _lrelu(mx + sc)


def _edge_specs(extra):
    ne = (_B * _N) // _EPTS
    specs = [
        pl.BlockSpec((_EPTS * _K, _COUT), lambda i: (i, 0)),
        pl.BlockSpec((_EPTS, _COUT), lambda i: (i, 0)),
        pl.BlockSpec((_EPTS, _K), lambda i: (i, 0)),
        pl.BlockSpec((1, _COUT), lambda i: (0, 0)),
    ]
    specs += [pl.BlockSpec(s, lambda i: (0, 0)) for s in extra]
    return ne, specs


def _stats1_pass(G, vT, dist, wd):
    ne, specs = _edge_specs([])
    return pl.pallas_call(
        _stats1_kernel, grid=(ne,), in_specs=specs,
        out_specs=pl.BlockSpec((2, _COUT), lambda i: (0, 0)),
        out_shape=jax.ShapeDtypeStruct((2, _COUT), jnp.float32),
    )(G, vT, dist, wd)


def _stats2_pass(G, vT, dist, wd, s2, t2, We2T):
    ne, specs = _edge_specs([(1, _COUT), (1, _COUT), (_COUT, _COUT)])
    return pl.pallas_call(
        _stats2_kernel, grid=(ne,), in_specs=specs,
        out_specs=pl.BlockSpec((2, _COUT), lambda i: (0, 0)),
        out_shape=jax.ShapeDtypeStruct((2, _COUT), jnp.float32),
    )(G, vT, dist, wd, s2, t2, We2T)


def _final_pass(G, vT, dist, wd, s2, t2, We2T, s3, t3, SCpT, s4, t4):
    ne, specs = _edge_specs([(1, _COUT), (1, _COUT), (_COUT, _COUT),
                             (1, _COUT), (1, _COUT)])
    specs.append(pl.BlockSpec((_EPTS, _COUT), lambda i: (i, 0)))
    specs += [pl.BlockSpec((1, _COUT), lambda i: (0, 0))] * 2
    return pl.pallas_call(
        _final_kernel, grid=(ne,), in_specs=specs,
        out_specs=pl.BlockSpec((_EPTS, _COUT), lambda i: (i, 0)),
        out_shape=jax.ShapeDtypeStruct((_B * _N, _COUT), jnp.float32),
    )(G, vT, dist, wd, s2, t2, We2T, s3, t3, SCpT, s4, t4)


def _bn_coef(sums, m, g, b):
    mean = sums[:, 0] / m
    var = sums[:, 1] / m - mean * mean
    s = g / jnp.sqrt(var + _EPS)
    return s, b - mean * s


def _bn_coef_row(sums, m, g, b):
    mean = sums[0] / m
    var = sums[1] / m - mean * mean
    s = g / jnp.sqrt(var + _EPS)
    return s.reshape(1, _COUT), (b - mean * s).reshape(1, _COUT)


def kernel(xyz, features, W_pre, g1, b1, W_e1, g2, b2, W_e2, g3, b3,
           W_sc, g4, b4):
    xyzT = jnp.transpose(xyz, (0, 2, 1))

    P, SCp, psums, ssums = _pre_pass(features, W_pre, W_sc)
    s1, t1 = _bn_coef(psums, _B * _N, g1, b1)
    s4r = (g4 / jnp.sqrt(ssums[:, 1] / (_B * _N)
                         - (ssums[:, 0] / (_B * _N)) ** 2 + _EPS))
    t4r = b4 - (ssums[:, 0] / (_B * _N)) * s4r
    s4 = s4r.reshape(1, _COUT)
    t4 = t4r.reshape(1, _COUT)

    idxf, dist = _knn_pass(xyz, xyzT)
    idx_flat = idxf[:, :, 1:].reshape(-1)
    dist16 = dist[:, :, 1:].reshape(_B * _N, _K)

    Wc = W_e1[:, :_HID]
    Wn = W_e1[:, _HID:2 * _HID]
    Wx = W_e1[:, 2 * _HID:2 * _HID + 3]
    wd = W_e1[:, 2 * _HID + 3].reshape(1, _COUT)
    u, v = _fuv_pass(P, xyzT, Wn, Wc - Wn, Wx,
                     s1.reshape(_HID, 1), t1.reshape(_HID, 1))

    uT = u.transpose(0, 2, 1).reshape(_B * _N, _COUT)
    vT = v.transpose(0, 2, 1).reshape(_B * _N, _COUT)
    SCpT = SCp.transpose(0, 2, 1).reshape(_B * _N, _COUT)

    G = _gather_rows(uT, idx_flat)

    M = _B * _N * _K
    sums1 = _stats1_pass(G, vT, dist16, wd)
    s2, t2 = _bn_coef_row(sums1, M, g2, b2)
    sums2 = _stats2_pass(G, vT, dist16, wd, s2, t2, W_e2.T)
    s3, t3 = _bn_coef_row(sums2, M, g3, b3)

    out = _final_pass(G, vT, dist16, wd, s2, t2, W_e2.T, s3, t3, SCpT, s4, t4)
    return out.reshape(_B, _N, _COUT).transpose(0, 2, 1)
